# Initial kernel scaffold; baseline (speedup 1.0000x reference)
import jax
import jax.numpy as jnp
from jax.experimental import pallas as pl


def kernel(x, edge_index, W_conv, b_conv, W_reg, b_reg):
    raise NotImplementedError("write your pallas kernel here")


# speedup vs baseline: 30.1573x; 30.1573x over previous
import functools

import jax
import jax.numpy as jnp
from jax import lax
from jax.experimental import pallas as pl
from jax.experimental.pallas import tpu as pltpu
from jax.experimental.pallas import tpu_sc as plsc

NC = 2
NS = 16
NW = NC * NS
K = 128


def _mesh():
    return plsc.VectorSubcoreMesh(
        core_axis_name="c", subcore_axis_name="s", num_cores=NC, num_subcores=NS
    )


def _row_chunks(rpt):
    chunks = [K] * (rpt // K)
    if rpt % K:
        chunks.append(rpt % K)
    return chunks


def _deg_kernel(n_acc, nch):
    rpt = n_acc // NS

    @functools.partial(
        pl.kernel,
        out_type=jax.ShapeDtypeStruct((NC * n_acc,), jnp.float32),
        mesh=_mesh(),
        scratch_types=[
            pltpu.VMEM((nch, K), jnp.int32),
            pltpu.VMEM((K,), jnp.float32),
            pltpu.VMEM((rpt,), jnp.float32),
            pltpu.VMEM_SHARED((n_acc,), jnp.float32),
            pltpu.SemaphoreType.DMA,
        ],
    )
    def deg_k(dst_hbm, ones_hbm, zeros1_hbm, out_hbm, idx_v, ones_v, zbuf, acc,
              sem):
        c = lax.axis_index("c")
        s = lax.axis_index("s")
        wid = s * NC + c
        r0 = s * rpt
        pltpu.sync_copy(zeros1_hbm, zbuf)
        pltpu.sync_copy(zbuf, acc.at[pl.ds(r0, rpt)])
        pltpu.sync_copy(ones_hbm, ones_v)
        pltpu.sync_copy(dst_hbm.at[wid], idx_v)
        plsc.subcore_barrier()

        def body(j, carry):
            pltpu.sync_copy(ones_v, acc.at[idx_v.at[j]], add=True)
            return carry

        lax.fori_loop(0, nch, body, 0)
        plsc.subcore_barrier()
        pltpu.sync_copy(acc.at[pl.ds(r0, rpt)], zbuf)
        pltpu.sync_copy(zbuf, out_hbm.at[pl.ds(c * n_acc + r0, rpt)])

    return deg_k


def _scatter_kernel(n, d, n_acc, nch):
    rpt = n_acc // NS

    @functools.partial(
        pl.kernel,
        out_type=jax.ShapeDtypeStruct((NC, n_acc, d), jnp.float32),
        mesh=_mesh(),
        scratch_types=[
            pltpu.VMEM((nch, K), jnp.int32),
            pltpu.VMEM((nch, K), jnp.int32),
            pltpu.VMEM((K, d), jnp.float32),
            pltpu.VMEM_SHARED((n_acc, d), jnp.float32),
            pltpu.SemaphoreType.DMA,
        ],
    )
    def scat_k(y_hbm, src_hbm, dst_hbm, zeros2_hbm, out_hbm,
               srcv, dstv, buf, acc, gsem):
        c = lax.axis_index("c")
        s = lax.axis_index("s")
        wid = s * NC + c
        r0 = s * rpt
        pltpu.sync_copy(zeros2_hbm, buf)
        off = 0
        for ck in _row_chunks(rpt):
            pltpu.sync_copy(buf.at[pl.ds(0, ck)], acc.at[pl.ds(r0 + off, ck)])
            off += ck
        pltpu.sync_copy(src_hbm.at[wid], srcv)
        pltpu.sync_copy(dst_hbm.at[wid], dstv)
        plsc.subcore_barrier()

        def body(j, carry):
            pltpu.async_copy(y_hbm.at[srcv.at[j]], buf, gsem).wait()
            pltpu.sync_copy(buf, acc.at[dstv.at[j]], add=True)
            return carry

        lax.fori_loop(0, nch, body, 0)
        plsc.subcore_barrier()
        off = 0
        for ck in _row_chunks(rpt):
            pltpu.sync_copy(acc.at[pl.ds(r0 + off, ck)], buf.at[pl.ds(0, ck)])
            pltpu.sync_copy(
                buf.at[pl.ds(0, ck)], out_hbm.at[c, pl.ds(r0 + off, ck)]
            )
            off += ck

    return scat_k


def kernel(x, edge_index, W_conv, b_conv, W_reg, b_reg):
    n, d = x.shape
    e = edge_index.shape[1]

    ew = -(-e // (NW * K)) * K
    if (ew // K) % 2:
        ew += K
    nch = ew // K
    e_pad = ew * NW
    npad = e_pad - e
    n_acc = -(-n // (8 * NS)) * (8 * NS)
    if n_acc - n < NS:
        n_acc += 8 * NS
    rpt = n_acc // NS

    src = edge_index[0]
    dst = edge_index[1]
    pad_i = jnp.arange(npad, dtype=jnp.int32)
    src_p = jnp.concatenate([src, pad_i % n])
    dst_p = jnp.concatenate([dst, n + pad_i % (n_acc - n)])
    src3 = src_p.reshape(NW, nch, K)
    dst3 = dst_p.reshape(NW, nch, K)

    ones_k = jnp.ones((K,), jnp.float32)
    zeros1 = jnp.zeros((rpt,), jnp.float32)
    zeros2 = jnp.zeros((K, d), jnp.float32)

    degp = _deg_kernel(n_acc, nch)(dst3, ones_k, zeros1)
    degt = degp.reshape(NC, n_acc).T

    bn = 1024
    gb = -(-n // bn)

    def y_body(x_ref, w_ref, degt_ref, y_ref):
        ds_ = degt_ref[...]
        dinv = lax.rsqrt(ds_[:, 0:1] + ds_[:, 1:2] + 1.0)
        xw = jnp.dot(x_ref[...], w_ref[...], preferred_element_type=jnp.float32)
        y_ref[...] = xw * dinv

    y = pl.pallas_call(
        y_body,
        grid=(gb,),
        in_specs=[
            pl.BlockSpec((bn, d), lambda j: (j, 0)),
            pl.BlockSpec((d, d), lambda j: (0, 0)),
            pl.BlockSpec((bn, NC), lambda j: (j, 0)),
        ],
        out_specs=pl.BlockSpec((bn, d), lambda j: (j, 0)),
        out_shape=jax.ShapeDtypeStruct((n, d), jnp.float32),
    )(x, W_conv, degt)

    sp = _scatter_kernel(n, d, n_acc, nch)(y, src3, dst3, zeros2)

    b_conv2 = b_conv.reshape(1, d)
    b_reg2 = b_reg.reshape(1, 1)

    def head_body(s_ref, y_ref, degt_ref, bc_ref, wr_ref, br_ref, o_ref):
        ds_ = degt_ref[...]
        dinv = lax.rsqrt(ds_[:, 0:1] + ds_[:, 1:2] + 1.0)
        tot = s_ref[0] + s_ref[1] + y_ref[...]
        agg = tot * dinv + bc_ref[...]
        h = jnp.maximum(agg, 0.0)
        o_ref[...] = (
            jnp.dot(h, wr_ref[...], preferred_element_type=jnp.float32)
            + br_ref[...]
        )

    out = pl.pallas_call(
        head_body,
        grid=(gb,),
        in_specs=[
            pl.BlockSpec((NC, bn, d), lambda j: (0, j, 0)),
            pl.BlockSpec((bn, d), lambda j: (j, 0)),
            pl.BlockSpec((bn, NC), lambda j: (j, 0)),
            pl.BlockSpec((1, d), lambda j: (0, 0)),
            pl.BlockSpec((d, 1), lambda j: (0, 0)),
            pl.BlockSpec((1, 1), lambda j: (0, 0)),
        ],
        out_specs=pl.BlockSpec((bn, 1), lambda j: (j, 0)),
        out_shape=jax.ShapeDtypeStruct((n, 1), jnp.float32),
    )(sp, y, degt, b_conv2, W_reg, b_reg2)

    return out


# speedup vs baseline: 39.7169x; 1.3170x over previous
import functools

import jax
import jax.numpy as jnp
from jax import lax
from jax.experimental import pallas as pl
from jax.experimental.pallas import tpu as pltpu
from jax.experimental.pallas import tpu_sc as plsc

NC = 2
NS = 16
NW = NC * NS


def _mesh():
    return plsc.VectorSubcoreMesh(
        core_axis_name="c", subcore_axis_name="s", num_cores=NC, num_subcores=NS
    )


def _row_chunks(rpt, k):
    chunks = [k] * (rpt // k)
    if rpt % k:
        chunks.append(rpt % k)
    return chunks


def _deg_kernel(n_acc, nch, k):
    rpt = n_acc // NS

    @functools.partial(
        pl.kernel,
        out_type=jax.ShapeDtypeStruct((NC * n_acc,), jnp.float32),
        mesh=_mesh(),
        scratch_types=[
            pltpu.VMEM((nch, k), jnp.int32),
            pltpu.VMEM((128,), jnp.float32),
            pltpu.VMEM((rpt,), jnp.float32),
            pltpu.VMEM_SHARED((n_acc,), jnp.float32),
            pltpu.SemaphoreType.DMA,
        ],
    )
    def deg_k(dst_hbm, ones_hbm, zeros1_hbm, out_hbm, idx_v, ones_v, zbuf, acc,
              sem):
        c = lax.axis_index("c")
        s = lax.axis_index("s")
        wid = s * NC + c
        r0 = s * rpt
        pltpu.sync_copy(zeros1_hbm, zbuf)
        pltpu.sync_copy(zbuf, acc.at[pl.ds(r0, rpt)])
        pltpu.sync_copy(ones_hbm, ones_v)
        pltpu.sync_copy(dst_hbm.at[wid], idx_v)
        plsc.subcore_barrier()

        ones_src = ones_v.at[pl.ds(0, k)] if k != 128 else ones_v

        def body(j, carry):
            pltpu.sync_copy(ones_src, acc.at[idx_v.at[j]], add=True)
            return carry

        lax.fori_loop(0, nch, body, 0)
        plsc.subcore_barrier()
        pltpu.sync_copy(acc.at[pl.ds(r0, rpt)], zbuf)
        pltpu.sync_copy(zbuf, out_hbm.at[pl.ds(c * n_acc + r0, rpt)])

    return deg_k


def _scatter_kernel(n, d, n_acc, nch, k):
    rpt = n_acc // NS
    G = 16
    assert nch % G == 0

    @functools.partial(
        pl.kernel,
        out_type=jax.ShapeDtypeStruct((NC, n_acc, d), jnp.float32),
        mesh=_mesh(),
        scratch_types=[
            pltpu.VMEM((G, k), jnp.int32),
            pltpu.VMEM((G, k), jnp.int32),
            [pltpu.VMEM((128, d), jnp.float32)] * 2,
            pltpu.VMEM_SHARED((n_acc, d), jnp.float32),
            [pltpu.SemaphoreType.DMA] * 2,
            [pltpu.SemaphoreType.DMA] * 2,
        ],
    )
    def scat_k(y_hbm, src_hbm, dst_hbm, zeros2_hbm, out_hbm,
               sidx, didx, bufs, acc, gsems, ssems):
        c = lax.axis_index("c")
        s = lax.axis_index("s")
        wid = s * NC + c
        r0 = s * rpt
        dbufs = [b_.at[pl.ds(0, k)] if k != 128 else b_ for b_ in bufs]
        pltpu.sync_copy(zeros2_hbm, bufs[0])
        off = 0
        for ck in _row_chunks(rpt, 128):
            pltpu.sync_copy(
                bufs[0].at[pl.ds(0, ck)], acc.at[pl.ds(r0 + off, ck)]
            )
            off += ck
        plsc.subcore_barrier()

        def group_body(g, carry):
            pltpu.sync_copy(src_hbm.at[wid, pl.ds(g * G, G)], sidx)
            pltpu.sync_copy(dst_hbm.at[wid, pl.ds(g * G, G)], didx)
            pltpu.async_copy(y_hbm.at[sidx.at[0]], dbufs[0], gsems[0])
            for t in range(G):
                b = t % 2
                bo = 1 - b
                if t + 1 < G:
                    if t >= 1:
                        pltpu.make_async_copy(
                            dbufs[bo], acc.at[didx.at[t - 1]], ssems[bo]
                        ).wait()
                    pltpu.async_copy(
                        y_hbm.at[sidx.at[t + 1]], dbufs[bo], gsems[bo]
                    )
                pltpu.make_async_copy(
                    y_hbm.at[sidx.at[t]], dbufs[b], gsems[b]
                ).wait()
                pltpu.async_copy(
                    dbufs[b], acc.at[didx.at[t]], ssems[b], add=True
                )
            for t in (G - 2, G - 1):
                pltpu.make_async_copy(
                    dbufs[t % 2], acc.at[didx.at[t]], ssems[t % 2]
                ).wait()
            return carry

        lax.fori_loop(0, nch // G, group_body, 0)
        plsc.subcore_barrier()
        off = 0
        for ck in _row_chunks(rpt, 128):
            pltpu.sync_copy(
                acc.at[pl.ds(r0 + off, ck)], bufs[0].at[pl.ds(0, ck)]
            )
            pltpu.sync_copy(
                bufs[0].at[pl.ds(0, ck)], out_hbm.at[c, pl.ds(r0 + off, ck)]
            )
            off += ck

    return scat_k


def kernel(x, edge_index, W_conv, b_conv, W_reg, b_reg):
    n, d = x.shape
    e = edge_index.shape[1]

    ew = e // NW
    assert ew * NW == e
    k = 125 if ew % 125 == 0 else 128
    assert ew % k == 0
    nch = ew // k
    n_acc = -(-n // (8 * NS)) * (8 * NS)
    rpt = n_acc // NS

    src3 = edge_index[0].reshape(NW, nch, k)
    dst3 = edge_index[1].reshape(NW, nch, k)

    ones_k = jnp.ones((128,), jnp.float32)
    zeros1 = jnp.zeros((rpt,), jnp.float32)
    zeros2 = jnp.zeros((128, d), jnp.float32)

    bn = 2048
    gb = -(-n // bn)

    def xw_body(x_ref, w_ref, o_ref):
        o_ref[...] = jnp.dot(
            x_ref[...], w_ref[...], preferred_element_type=jnp.float32
        )

    xw = pl.pallas_call(
        xw_body,
        grid=(gb,),
        in_specs=[
            pl.BlockSpec((bn, d), lambda j: (j, 0)),
            pl.BlockSpec((d, d), lambda j: (0, 0)),
        ],
        out_specs=pl.BlockSpec((bn, d), lambda j: (j, 0)),
        out_shape=jax.ShapeDtypeStruct((n, d), jnp.float32),
    )(x, W_conv)

    degp = _deg_kernel(n_acc, nch, k)(dst3, ones_k, zeros1)
    degt = degp.reshape(NC, n_acc).T

    def y_body(xw_ref, degt_ref, y_ref):
        ds_ = degt_ref[...]
        dinv = lax.rsqrt(ds_[:, 0:1] + ds_[:, 1:2] + 1.0)
        y_ref[...] = xw_ref[...] * dinv

    y = pl.pallas_call(
        y_body,
        grid=(gb,),
        in_specs=[
            pl.BlockSpec((bn, d), lambda j: (j, 0)),
            pl.BlockSpec((bn, NC), lambda j: (j, 0)),
        ],
        out_specs=pl.BlockSpec((bn, d), lambda j: (j, 0)),
        out_shape=jax.ShapeDtypeStruct((n, d), jnp.float32),
    )(xw, degt)

    sp = _scatter_kernel(n, d, n_acc, nch, k)(y, src3, dst3, zeros2)

    b_conv2 = b_conv.reshape(1, d)
    b_reg2 = b_reg.reshape(1, 1)

    def head_body(s_ref, y_ref, degt_ref, bc_ref, wr_ref, br_ref, o_ref):
        ds_ = degt_ref[...]
        dinv = lax.rsqrt(ds_[:, 0:1] + ds_[:, 1:2] + 1.0)
        tot = s_ref[0] + s_ref[1] + y_ref[...]
        agg = tot * dinv + bc_ref[...]
        h = jnp.maximum(agg, 0.0)
        o_ref[...] = (
            jnp.dot(h, wr_ref[...], preferred_element_type=jnp.float32)
            + br_ref[...]
        )

    out = pl.pallas_call(
        head_body,
        grid=(gb,),
        in_specs=[
            pl.BlockSpec((NC, bn, d), lambda j: (0, j, 0)),
            pl.BlockSpec((bn, d), lambda j: (j, 0)),
            pl.BlockSpec((bn, NC), lambda j: (j, 0)),
            pl.BlockSpec((1, d), lambda j: (0, 0)),
            pl.BlockSpec((d, 1), lambda j: (0, 0)),
            pl.BlockSpec((1, 1), lambda j: (0, 0)),
        ],
        out_specs=pl.BlockSpec((bn, 1), lambda j: (j, 0)),
        out_shape=jax.ShapeDtypeStruct((n, 1), jnp.float32),
    )(sp, y, degt, b_conv2, W_reg, b_reg2)

    return out
